# baseline (speedup 1.0000x reference)
import functools

import jax
import jax.numpy as jnp
from jax import lax
from jax.experimental import pallas as pl
from jax.experimental.pallas import tpu as pltpu
from jax.experimental.pallas import tpu_sc as plsc

N = 10000
NUM_GRAPHS = 128
NP = 10240
NTILES = 32
NBLK = 42
BE = 128
STRIPE = NP // 16
ZR = 64
TOT_E = NTILES * NBLK * BE
TR = 400
G = N // TR
F32 = jnp.float32

_MESH = plsc.VectorSubcoreMesh(core_axis_name="c", subcore_axis_name="s",
                               num_cores=2, num_subcores=16)


def _zero_fill(buf, rows, width=128):
  @pl.loop(0, rows)
  def _(r):
    @pl.loop(0, width, step=16)
    def _(f):
      buf[r, pl.ds(f, 16)] = jnp.zeros((16,), F32)


def _deg_body(dst_hbm, out_hbm, dstv, ones_v, zv, acc, sem):
  del sem
  c = lax.axis_index("c")
  s = lax.axis_index("s")
  wid = s * 2 + c
  pltpu.sync_copy(dst_hbm.at[wid], dstv)

  @pl.loop(0, BE)
  def _(r):
    @pl.loop(0, 128, step=16)
    def _(f):
      ones_v[r, pl.ds(f, 16)] = jnp.full((16,), 1.0, F32)

  _zero_fill(zv, ZR)
  base = s * STRIPE
  @pl.loop(0, STRIPE // ZR)
  def _(j):
    pltpu.sync_copy(zv, acc.at[pl.ds(base + j * ZR, ZR)])
  plsc.subcore_barrier()

  @pl.loop(0, NBLK)
  def _(blk):
    pltpu.sync_copy(ones_v, acc.at[dstv.at[blk]], add=True)
  plsc.subcore_barrier()
  pltpu.sync_copy(acc.at[pl.ds(base, STRIPE)], out_hbm.at[c, pl.ds(base, STRIPE)])


def _compute_deg(dst_idx):
  k = pl.kernel(
      _deg_body,
      out_type=jax.ShapeDtypeStruct((2, NP, 128), F32),
      mesh=_MESH,
      scratch_types=[
          pltpu.VMEM((NBLK, BE), jnp.int32),
          pltpu.VMEM((BE, 128), F32),
          pltpu.VMEM((ZR, 128), F32),
          pltpu.VMEM_SHARED((NP, 128), F32),
          pltpu.SemaphoreType.DMA,
      ],
  )
  return k(dst_idx)


def _make_prop_body(nchunks):
  def body(src_hbm, dst_hbm, *rest):
    tabs = rest[:nchunks]
    out_hbm = rest[nchunks]
    srcv, dstv, g0, g1, sem0, sem1, sems0, sems1, acc = rest[nchunks + 1:]
    c = lax.axis_index("c")
    s = lax.axis_index("s")
    wid = s * 2 + c
    pltpu.sync_copy(src_hbm.at[wid], srcv)
    pltpu.sync_copy(dst_hbm.at[wid], dstv)
    base = s * STRIPE

    for kk in range(nchunks):
      tab = tabs[kk]

      _zero_fill(g0, BE)

      @pl.loop(0, STRIPE // BE)
      def _(j):
        pltpu.sync_copy(g0, acc.at[pl.ds(base + j * BE, BE)])
      plsc.subcore_barrier()

      pltpu.async_copy(tab.at[srcv.at[0]], g0, sem0)
      pltpu.async_copy(tab.at[srcv.at[1]], g1, sem1)

      @pl.loop(0, NBLK, step=2)
      def _(blk):
        pltpu.make_async_copy(tab.at[srcv.at[blk]], g0, sem0).wait()
        pltpu.async_copy(g0, acc.at[dstv.at[blk]], sems0, add=True)

        pltpu.make_async_copy(tab.at[srcv.at[blk + 1]], g1, sem1).wait()
        pltpu.async_copy(g1, acc.at[dstv.at[blk + 1]], sems1, add=True)

        pltpu.make_async_copy(g0, acc.at[dstv.at[blk]], sems0).wait()

        @pl.when(blk + 2 < NBLK)
        def _():
          pltpu.async_copy(tab.at[srcv.at[blk + 2]], g0, sem0)

        pltpu.make_async_copy(g1, acc.at[dstv.at[blk + 1]], sems1).wait()

        @pl.when(blk + 3 < NBLK)
        def _():
          pltpu.async_copy(tab.at[srcv.at[blk + 3]], g1, sem1)

      plsc.subcore_barrier()
      pltpu.sync_copy(acc.at[pl.ds(base, STRIPE)],
                      out_hbm.at[c, kk, pl.ds(base, STRIPE)])
  return body


def _prop(src_idx, dst_idx, tabs):
  nchunks = len(tabs)
  k = pl.kernel(
      _make_prop_body(nchunks),
      out_type=jax.ShapeDtypeStruct((2, nchunks, NP, 128), F32),
      mesh=_MESH,
      scratch_types=[
          pltpu.VMEM((NBLK, BE), jnp.int32),
          pltpu.VMEM((NBLK, BE), jnp.int32),
          pltpu.VMEM((BE, 128), F32),
          pltpu.VMEM((BE, 128), F32),
          pltpu.SemaphoreType.DMA,
          pltpu.SemaphoreType.DMA,
          pltpu.SemaphoreType.DMA,
          pltpu.SemaphoreType.DMA,
          pltpu.VMEM_SHARED((NP, 128), F32),
      ],
  )
  return k(src_idx, dst_idx, *tabs)


def _dinv_of(deg_blk):
  d = deg_blk[0, :, 0:1] + deg_blk[1, :, 0:1]
  return lax.rsqrt(d)


def _mpre_body(x_ref, deg_ref, o_ref):
  o_ref[...] = x_ref[...] * _dinv_of(deg_ref[...])


def _m12_body(p0_ref, deg_ref, w1_ref, b1_ref, w2_ref, *outs):
  dinv = _dinv_of(deg_ref[...])
  p0 = p0_ref[...]
  p = ((p0[0, 0] + p0[1, 0]) * dinv).astype(jnp.bfloat16)
  h = jnp.maximum(
      jnp.dot(p, w1_ref[...].astype(jnp.bfloat16),
              preferred_element_type=F32) + b1_ref[...], 0.0)
  t = jnp.dot(h.astype(jnp.bfloat16), w2_ref[...].astype(jnp.bfloat16),
              preferred_element_type=F32) * dinv
  for kk in range(6):
    outs[kk][...] = t[:, 128 * kk:128 * (kk + 1)]


def _m3_body(p1_ref, deg_ref, b2_ref, w3_ref, oa_ref, ob_ref):
  dinv = _dinv_of(deg_ref[...])
  p1 = p1_ref[...]
  pc = jnp.concatenate([p1[0, kk] + p1[1, kk] for kk in range(6)], axis=1)
  h = jnp.maximum(pc * dinv + b2_ref[...], 0.0)
  t = jnp.dot(h, w3_ref[...], preferred_element_type=F32) * dinv
  oa_ref[...] = t[:, 0:128]
  ob_ref[...] = t[:, 128:256]


def _m4_body(p2_ref, deg_ref, b3_ref, batch_ref, wl_ref, bl_ref, out_ref,
             sums, cnts):
  i = pl.program_id(0)

  @pl.when(i == 0)
  def _():
    sums[...] = jnp.zeros_like(sums)
    cnts[...] = jnp.zeros_like(cnts)

  dinv = _dinv_of(deg_ref[...])
  p2 = p2_ref[...]
  pc = jnp.concatenate([p2[0, 0] + p2[1, 0], p2[0, 1] + p2[1, 1]], axis=1)
  h3 = jnp.maximum(pc * dinv + b3_ref[...], 0.0)
  gf = (batch_ref[...] == lax.broadcasted_iota(
      jnp.int32, (TR, NUM_GRAPHS), 1).astype(F32)).astype(F32)
  dn = (((0,), (0,)), ((), ()))
  sums[...] += lax.dot_general(gf, h3, dn, preferred_element_type=F32)
  cnts[...] += lax.dot_general(gf, jnp.ones((TR, 8), F32), dn,
                               preferred_element_type=F32)

  @pl.when(i == G - 1)
  def _():
    pooled = sums[...] / jnp.maximum(cnts[...][:, 0:1], 1.0)
    out_ref[...] = (jnp.dot(pooled, wl_ref[...], preferred_element_type=F32)
                    + bl_ref[...])


def _full(shape):
  return pl.BlockSpec(shape, lambda i: tuple(0 for _ in shape))


@jax.jit
def kernel(x, edge_index, batch, W1, b1, W2, b2, W3, b3, Wl, bl):
  ei = edge_index.astype(jnp.int32)
  loop = jnp.arange(N, dtype=jnp.int32)
  npad = TOT_E - (ei.shape[1] + N)
  padv = jnp.arange(npad, dtype=jnp.int32) % 16
  src = jnp.concatenate([ei[0], loop, padv]).reshape(NTILES, NBLK, BE)
  dst = jnp.concatenate([ei[1], loop, N + padv]).reshape(NTILES, NBLK, BE)

  w2p = jnp.pad(W2, ((0, 0), (0, 68)))
  b2p = jnp.pad(b2, (0, 68)).reshape(1, 768)
  w3p = jnp.pad(W3, ((0, 68), (0, 56)))
  b3p = jnp.pad(b3, (0, 56)).reshape(1, 256)
  wlp = jnp.pad(Wl, ((0, 56), (0, 0)))
  blp = bl.reshape(1, 10)
  b1p = b1.reshape(1, 1000)
  batchf = jnp.broadcast_to(
      batch.astype(F32)[:, None], (N, NUM_GRAPHS))

  degp = _compute_deg(dst)

  deg_spec = pl.BlockSpec((2, TR, 128), lambda i: (0, i, 0))

  xt = pl.pallas_call(
      _mpre_body,
      grid=(G,),
      in_specs=[pl.BlockSpec((TR, 128), lambda i: (i, 0)), deg_spec],
      out_specs=pl.BlockSpec((TR, 128), lambda i: (i, 0)),
      out_shape=jax.ShapeDtypeStruct((N, 128), F32),
  )(x, degp)

  p0 = _prop(src, dst, [xt])

  t1c = pl.pallas_call(
      _m12_body,
      grid=(G,),
      in_specs=[
          pl.BlockSpec((2, 1, TR, 128), lambda i: (0, 0, i, 0)),
          deg_spec,
          _full((128, 1000)),
          _full((1, 1000)),
          _full((1000, 768)),
      ],
      out_specs=[pl.BlockSpec((TR, 128), lambda i: (i, 0))] * 6,
      out_shape=[jax.ShapeDtypeStruct((N, 128), F32)] * 6,
  )(p0, degp, W1, b1p, w2p)

  p1 = _prop(src, dst, list(t1c))

  t2c = pl.pallas_call(
      _m3_body,
      grid=(G,),
      in_specs=[
          pl.BlockSpec((2, 6, TR, 128), lambda i: (0, 0, i, 0)),
          deg_spec,
          _full((1, 768)),
          _full((768, 256)),
      ],
      out_specs=[pl.BlockSpec((TR, 128), lambda i: (i, 0))] * 2,
      out_shape=[jax.ShapeDtypeStruct((N, 128), F32)] * 2,
  )(p1, degp, b2p, w3p)

  p2 = _prop(src, dst, list(t2c))

  out = pl.pallas_call(
      _m4_body,
      grid=(G,),
      in_specs=[
          pl.BlockSpec((2, 2, TR, 128), lambda i: (0, 0, i, 0)),
          deg_spec,
          _full((1, 256)),
          pl.BlockSpec((TR, NUM_GRAPHS), lambda i: (i, 0)),
          _full((256, 10)),
          _full((1, 10)),
      ],
      out_specs=pl.BlockSpec((NUM_GRAPHS, 10), lambda i: (0, 0)),
      out_shape=jax.ShapeDtypeStruct((NUM_GRAPHS, 10), F32),
      scratch_shapes=[
          pltpu.VMEM((NUM_GRAPHS, 256), F32),
          pltpu.VMEM((NUM_GRAPHS, 8), F32),
      ],
  )(p2, degp, b3p, batchf, wlp, blp)

  return out


# speedup vs baseline: 1.2397x; 1.2397x over previous
import jax
import jax.numpy as jnp
from jax import lax
from jax.experimental import pallas as pl
from jax.experimental.pallas import tpu as pltpu
from jax.experimental.pallas import tpu_sc as plsc

N = 10000
NUM_GRAPHS = 128
NP = 10240
NTILES = 32
NBLK = 42
BE = 128
STRIPE = NP // 16
ZR = 64
TOT_E = NTILES * NBLK * BE
TR = 400
G = N // TR
F32 = jnp.float32

_MESH = plsc.VectorSubcoreMesh(core_axis_name="c", subcore_axis_name="s",
                               num_cores=2, num_subcores=16)


def _zero_fill(buf, rows, width=128):
  @pl.loop(0, rows)
  def _(r):
    @pl.loop(0, width, step=16)
    def _(f):
      buf[r, pl.ds(f, 16)] = jnp.zeros((16,), F32)


def _deg_body(dst_hbm, out_hbm, dstv, ones_v, zv, acc, sem):
  del sem
  c = lax.axis_index("c")
  s = lax.axis_index("s")
  wid = s * 2 + c
  pltpu.sync_copy(dst_hbm.at[wid], dstv)

  @pl.loop(0, BE)
  def _(r):
    @pl.loop(0, 128, step=16)
    def _(f):
      ones_v[r, pl.ds(f, 16)] = jnp.full((16,), 1.0, F32)

  _zero_fill(zv, ZR)
  base = s * STRIPE
  @pl.loop(0, STRIPE // ZR)
  def _(j):
    pltpu.sync_copy(zv, acc.at[pl.ds(base + j * ZR, ZR)])
  plsc.subcore_barrier()

  @pl.loop(0, NBLK)
  def _(blk):
    pltpu.sync_copy(ones_v, acc.at[dstv.at[blk]], add=True)
  plsc.subcore_barrier()
  pltpu.sync_copy(acc.at[pl.ds(base, STRIPE)], out_hbm.at[c, pl.ds(base, STRIPE)])


def _compute_deg(dst_idx):
  k = pl.kernel(
      _deg_body,
      out_type=jax.ShapeDtypeStruct((2, NP, 128), F32),
      mesh=_MESH,
      scratch_types=[
          pltpu.VMEM((NBLK, BE), jnp.int32),
          pltpu.VMEM((BE, 128), F32),
          pltpu.VMEM((ZR, 128), F32),
          pltpu.VMEM_SHARED((NP, 128), F32),
          pltpu.SemaphoreType.DMA,
      ],
  )
  return k(dst_idx)


def _make_prop_body(nchunks):
  def body(src_hbm, dst_hbm, *rest):
    tabs = rest[:nchunks]
    out_hbm = rest[nchunks]
    srcv, dstv, g0, g1, sem0, sem1, acc = rest[nchunks + 1:]
    c = lax.axis_index("c")
    s = lax.axis_index("s")
    wid = s * 2 + c
    pltpu.sync_copy(src_hbm.at[wid], srcv)
    pltpu.sync_copy(dst_hbm.at[wid], dstv)
    base = s * STRIPE

    for kk in range(nchunks):
      tab = tabs[kk]

      _zero_fill(g0, BE)

      @pl.loop(0, STRIPE // BE)
      def _(j):
        pltpu.sync_copy(g0, acc.at[pl.ds(base + j * BE, BE)])
      plsc.subcore_barrier()

      pltpu.async_copy(tab.at[srcv.at[0]], g0, sem0)
      pltpu.async_copy(tab.at[srcv.at[1]], g1, sem1)

      @pl.loop(0, NBLK, step=2)
      def _(blk):
        pltpu.make_async_copy(tab.at[srcv.at[blk]], g0, sem0).wait()
        pltpu.sync_copy(g0, acc.at[dstv.at[blk]], add=True)

        @pl.when(blk + 2 < NBLK)
        def _():
          pltpu.async_copy(tab.at[srcv.at[blk + 2]], g0, sem0)

        pltpu.make_async_copy(tab.at[srcv.at[blk + 1]], g1, sem1).wait()
        pltpu.sync_copy(g1, acc.at[dstv.at[blk + 1]], add=True)

        @pl.when(blk + 3 < NBLK)
        def _():
          pltpu.async_copy(tab.at[srcv.at[blk + 3]], g1, sem1)

      plsc.subcore_barrier()
      pltpu.sync_copy(acc.at[pl.ds(base, STRIPE)],
                      out_hbm.at[c, kk, pl.ds(base, STRIPE)])
  return body


def _prop(src_idx, dst_idx, tabs):
  nchunks = len(tabs)
  k = pl.kernel(
      _make_prop_body(nchunks),
      out_type=jax.ShapeDtypeStruct((2, nchunks, NP, 128), F32),
      mesh=_MESH,
      scratch_types=[
          pltpu.VMEM((NBLK, BE), jnp.int32),
          pltpu.VMEM((NBLK, BE), jnp.int32),
          pltpu.VMEM((BE, 128), F32),
          pltpu.VMEM((BE, 128), F32),
          pltpu.SemaphoreType.DMA,
          pltpu.SemaphoreType.DMA,
          pltpu.VMEM_SHARED((NP, 128), F32),
      ],
  )
  return k(src_idx, dst_idx, *tabs)


def _dinv_of(deg_blk):
  d = deg_blk[0, :, 0:1] + deg_blk[1, :, 0:1]
  return lax.rsqrt(d)


def _mpre_body(x_ref, deg_ref, o_ref):
  o_ref[...] = x_ref[...] * _dinv_of(deg_ref[...])


def _m12_body(p0_ref, deg_ref, w1_ref, b1_ref, w2_ref, *outs):
  dinv = _dinv_of(deg_ref[...])
  p0 = p0_ref[...]
  p = (p0[0, 0] + p0[1, 0]) * dinv
  h = jnp.maximum(
      jnp.dot(p, w1_ref[...], preferred_element_type=F32) + b1_ref[...], 0.0)
  t = jnp.dot(h, w2_ref[...], preferred_element_type=F32) * dinv
  for kk in range(6):
    outs[kk][...] = t[:, 128 * kk:128 * (kk + 1)]


def _m3_body(p1_ref, deg_ref, b2_ref, w3_ref, oa_ref, ob_ref):
  dinv = _dinv_of(deg_ref[...])
  p1 = p1_ref[...]
  pc = jnp.concatenate([p1[0, kk] + p1[1, kk] for kk in range(6)], axis=1)
  h = jnp.maximum(pc * dinv + b2_ref[...], 0.0)
  t = jnp.dot(h, w3_ref[...], preferred_element_type=F32) * dinv
  oa_ref[...] = t[:, 0:128]
  ob_ref[...] = t[:, 128:256]


def _m4_body(p2_ref, deg_ref, b3_ref, batch_ref, wl_ref, bl_ref, out_ref,
             sums, cnts):
  i = pl.program_id(0)

  @pl.when(i == 0)
  def _():
    sums[...] = jnp.zeros_like(sums)
    cnts[...] = jnp.zeros_like(cnts)

  dinv = _dinv_of(deg_ref[...])
  p2 = p2_ref[...]
  pc = jnp.concatenate([p2[0, 0] + p2[1, 0], p2[0, 1] + p2[1, 1]], axis=1)
  h3 = jnp.maximum(pc * dinv + b3_ref[...], 0.0)
  gf = (batch_ref[...] == lax.broadcasted_iota(
      jnp.int32, (TR, NUM_GRAPHS), 1).astype(F32)).astype(F32)
  dn = (((0,), (0,)), ((), ()))
  sums[...] += lax.dot_general(gf, h3, dn, preferred_element_type=F32)
  cnts[...] += lax.dot_general(gf, jnp.ones((TR, 8), F32), dn,
                               preferred_element_type=F32)

  @pl.when(i == G - 1)
  def _():
    pooled = sums[...] / jnp.maximum(cnts[...][:, 0:1], 1.0)
    out_ref[...] = (jnp.dot(pooled, wl_ref[...], preferred_element_type=F32)
                    + bl_ref[...])


def _full(shape):
  return pl.BlockSpec(shape, lambda i: tuple(0 for _ in shape))


@jax.jit
def kernel(x, edge_index, batch, W1, b1, W2, b2, W3, b3, Wl, bl):
  ei = edge_index.astype(jnp.int32)
  loop = jnp.arange(N, dtype=jnp.int32)
  npad = TOT_E - (ei.shape[1] + N)
  padv = jnp.arange(npad, dtype=jnp.int32) % 16
  src = jnp.concatenate([ei[0], loop, padv]).reshape(NTILES, NBLK, BE)
  dst = jnp.concatenate([ei[1], loop, N + padv]).reshape(NTILES, NBLK, BE)

  w2p = jnp.pad(W2, ((0, 0), (0, 68)))
  b2p = jnp.pad(b2, (0, 68)).reshape(1, 768)
  w3p = jnp.pad(W3, ((0, 68), (0, 56)))
  b3p = jnp.pad(b3, (0, 56)).reshape(1, 256)
  wlp = jnp.pad(Wl, ((0, 56), (0, 0)))
  blp = bl.reshape(1, 10)
  b1p = b1.reshape(1, 1000)
  batchf = jnp.broadcast_to(
      batch.astype(F32)[:, None], (N, NUM_GRAPHS))

  degp = _compute_deg(dst)

  deg_spec = pl.BlockSpec((2, TR, 128), lambda i: (0, i, 0))

  xt = pl.pallas_call(
      _mpre_body,
      grid=(G,),
      in_specs=[pl.BlockSpec((TR, 128), lambda i: (i, 0)), deg_spec],
      out_specs=pl.BlockSpec((TR, 128), lambda i: (i, 0)),
      out_shape=jax.ShapeDtypeStruct((N, 128), F32),
  )(x, degp)

  p0 = _prop(src, dst, [xt])

  t1c = pl.pallas_call(
      _m12_body,
      grid=(G,),
      in_specs=[
          pl.BlockSpec((2, 1, TR, 128), lambda i: (0, 0, i, 0)),
          deg_spec,
          _full((128, 1000)),
          _full((1, 1000)),
          _full((1000, 768)),
      ],
      out_specs=[pl.BlockSpec((TR, 128), lambda i: (i, 0))] * 6,
      out_shape=[jax.ShapeDtypeStruct((N, 128), F32)] * 6,
  )(p0, degp, W1, b1p, w2p)

  p1 = _prop(src, dst, list(t1c))

  t2c = pl.pallas_call(
      _m3_body,
      grid=(G,),
      in_specs=[
          pl.BlockSpec((2, 6, TR, 128), lambda i: (0, 0, i, 0)),
          deg_spec,
          _full((1, 768)),
          _full((768, 256)),
      ],
      out_specs=[pl.BlockSpec((TR, 128), lambda i: (i, 0))] * 2,
      out_shape=[jax.ShapeDtypeStruct((N, 128), F32)] * 2,
  )(p1, degp, b2p, w3p)

  p2 = _prop(src, dst, list(t2c))

  out = pl.pallas_call(
      _m4_body,
      grid=(G,),
      in_specs=[
          pl.BlockSpec((2, 2, TR, 128), lambda i: (0, 0, i, 0)),
          deg_spec,
          _full((1, 256)),
          pl.BlockSpec((TR, NUM_GRAPHS), lambda i: (i, 0)),
          _full((256, 10)),
          _full((1, 10)),
      ],
      out_specs=pl.BlockSpec((NUM_GRAPHS, 10), lambda i: (0, 0)),
      out_shape=jax.ShapeDtypeStruct((NUM_GRAPHS, 10), F32),
      scratch_shapes=[
          pltpu.VMEM((NUM_GRAPHS, 256), F32),
          pltpu.VMEM((NUM_GRAPHS, 8), F32),
      ],
  )(p2, degp, b3p, batchf, wlp, blp)

  return out


# speedup vs baseline: 1.2951x; 1.0447x over previous
import jax
import jax.numpy as jnp
from jax import lax
from jax.experimental import pallas as pl
from jax.experimental.pallas import tpu as pltpu
from jax.experimental.pallas import tpu_sc as plsc

N = 10000
NUM_GRAPHS = 128
NP = 10240
NTILES = 32
NBLK = 42
BE = 128
STRIPE = NP // 16
ZR = 64
TOT_E = NTILES * NBLK * BE
TR = 1000
G = N // TR
F32 = jnp.float32

_MESH = plsc.VectorSubcoreMesh(core_axis_name="c", subcore_axis_name="s",
                               num_cores=2, num_subcores=16)


def _zero_fill(buf, rows, width=128):
  @pl.loop(0, rows)
  def _(r):
    @pl.loop(0, width, step=16)
    def _(f):
      buf[r, pl.ds(f, 16)] = jnp.zeros((16,), F32)


def _deg_body(dst_hbm, out_hbm, dstv, ones_v, zv, acc, sem):
  del sem
  c = lax.axis_index("c")
  s = lax.axis_index("s")
  wid = s * 2 + c
  pltpu.sync_copy(dst_hbm.at[wid], dstv)

  @pl.loop(0, BE)
  def _(r):
    @pl.loop(0, 128, step=16)
    def _(f):
      ones_v[r, pl.ds(f, 16)] = jnp.full((16,), 1.0, F32)

  _zero_fill(zv, ZR)
  base = s * STRIPE
  @pl.loop(0, STRIPE // ZR)
  def _(j):
    pltpu.sync_copy(zv, acc.at[pl.ds(base + j * ZR, ZR)])
  plsc.subcore_barrier()

  @pl.loop(0, NBLK)
  def _(blk):
    pltpu.sync_copy(ones_v, acc.at[dstv.at[blk]], add=True)
  plsc.subcore_barrier()
  pltpu.sync_copy(acc.at[pl.ds(base, STRIPE)], out_hbm.at[c, pl.ds(base, STRIPE)])


def _compute_deg(dst_idx):
  k = pl.kernel(
      _deg_body,
      out_type=jax.ShapeDtypeStruct((2, NP, 128), F32),
      mesh=_MESH,
      scratch_types=[
          pltpu.VMEM((NBLK, BE), jnp.int32),
          pltpu.VMEM((BE, 128), F32),
          pltpu.VMEM((ZR, 128), F32),
          pltpu.VMEM_SHARED((NP, 128), F32),
          pltpu.SemaphoreType.DMA,
      ],
  )
  return k(dst_idx)


def _make_prop_body(nchunks):
  def body(src_hbm, dst_hbm, *rest):
    tabs = rest[:nchunks]
    out_hbm = rest[nchunks]
    srcv, dstv, g0, g1, sem0, sem1, acc = rest[nchunks + 1:]
    c = lax.axis_index("c")
    s = lax.axis_index("s")
    wid = s * 2 + c
    pltpu.sync_copy(src_hbm.at[wid], srcv)
    pltpu.sync_copy(dst_hbm.at[wid], dstv)
    base = s * STRIPE

    for kk in range(nchunks):
      tab = tabs[kk]

      _zero_fill(g0, BE)

      @pl.loop(0, STRIPE // BE)
      def _(j):
        pltpu.sync_copy(g0, acc.at[pl.ds(base + j * BE, BE)])
      plsc.subcore_barrier()

      pltpu.async_copy(tab.at[srcv.at[0]], g0, sem0)
      pltpu.async_copy(tab.at[srcv.at[1]], g1, sem1)

      @pl.loop(0, NBLK, step=2)
      def _(blk):
        pltpu.make_async_copy(tab.at[srcv.at[blk]], g0, sem0).wait()
        pltpu.sync_copy(g0, acc.at[dstv.at[blk]], add=True)

        @pl.when(blk + 2 < NBLK)
        def _():
          pltpu.async_copy(tab.at[srcv.at[blk + 2]], g0, sem0)

        pltpu.make_async_copy(tab.at[srcv.at[blk + 1]], g1, sem1).wait()
        pltpu.sync_copy(g1, acc.at[dstv.at[blk + 1]], add=True)

        @pl.when(blk + 3 < NBLK)
        def _():
          pltpu.async_copy(tab.at[srcv.at[blk + 3]], g1, sem1)

      plsc.subcore_barrier()
      pltpu.sync_copy(acc.at[pl.ds(base, STRIPE)],
                      out_hbm.at[c, kk, pl.ds(base, STRIPE)])
  return body


def _prop(src_idx, dst_idx, tabs):
  nchunks = len(tabs)
  k = pl.kernel(
      _make_prop_body(nchunks),
      out_type=jax.ShapeDtypeStruct((2, nchunks, NP, 128), F32),
      mesh=_MESH,
      scratch_types=[
          pltpu.VMEM((NBLK, BE), jnp.int32),
          pltpu.VMEM((NBLK, BE), jnp.int32),
          pltpu.VMEM((BE, 128), F32),
          pltpu.VMEM((BE, 128), F32),
          pltpu.SemaphoreType.DMA,
          pltpu.SemaphoreType.DMA,
          pltpu.VMEM_SHARED((NP, 128), F32),
      ],
  )
  return k(src_idx, dst_idx, *tabs)


def _dinv_of(deg_blk):
  d = deg_blk[0, :, 0:1] + deg_blk[1, :, 0:1]
  return lax.rsqrt(d)


def _mpre_body(x_ref, deg_ref, o_ref):
  o_ref[...] = x_ref[...] * _dinv_of(deg_ref[...])


def _m12_body(p0_ref, deg_ref, w1_ref, b1_ref, w2_ref, *outs):
  dinv = _dinv_of(deg_ref[...])
  p0 = p0_ref[...]
  p = (p0[0, 0] + p0[1, 0]) * dinv
  h = jnp.maximum(
      jnp.dot(p, w1_ref[...], preferred_element_type=F32) + b1_ref[...], 0.0)
  t = jnp.dot(h, w2_ref[...], preferred_element_type=F32) * dinv
  for kk in range(6):
    outs[kk][...] = t[:, 128 * kk:128 * (kk + 1)]


def _m3_body(p1_ref, deg_ref, b2_ref, w3_ref, oa_ref, ob_ref):
  dinv = _dinv_of(deg_ref[...])
  p1 = p1_ref[...]
  pc = jnp.concatenate([p1[0, kk] + p1[1, kk] for kk in range(6)], axis=1)
  h = jnp.maximum(pc * dinv + b2_ref[...], 0.0)
  t = jnp.dot(h, w3_ref[...], preferred_element_type=F32) * dinv
  oa_ref[...] = t[:, 0:128]
  ob_ref[...] = t[:, 128:256]


def _m4_body(p2_ref, deg_ref, b3_ref, batch_ref, wl_ref, bl_ref, out_ref,
             sums, cnts):
  i = pl.program_id(0)

  @pl.when(i == 0)
  def _():
    sums[...] = jnp.zeros_like(sums)
    cnts[...] = jnp.zeros_like(cnts)

  dinv = _dinv_of(deg_ref[...])
  p2 = p2_ref[...]
  pc = jnp.concatenate([p2[0, 0] + p2[1, 0], p2[0, 1] + p2[1, 1]], axis=1)
  h3 = jnp.maximum(pc * dinv + b3_ref[...], 0.0)
  gf = (batch_ref[...] == lax.broadcasted_iota(
      jnp.int32, (TR, NUM_GRAPHS), 1).astype(F32)).astype(F32)
  dn = (((0,), (0,)), ((), ()))
  sums[...] += lax.dot_general(gf, h3, dn, preferred_element_type=F32)
  cnts[...] += lax.dot_general(gf, jnp.ones((TR, 8), F32), dn,
                               preferred_element_type=F32)

  @pl.when(i == G - 1)
  def _():
    pooled = sums[...] / jnp.maximum(cnts[...][:, 0:1], 1.0)
    out_ref[...] = (jnp.dot(pooled, wl_ref[...], preferred_element_type=F32)
                    + bl_ref[...])


def _full(shape):
  return pl.BlockSpec(shape, lambda i: tuple(0 for _ in shape))


@jax.jit
def kernel(x, edge_index, batch, W1, b1, W2, b2, W3, b3, Wl, bl):
  ei = edge_index.astype(jnp.int32)
  loop = jnp.arange(N, dtype=jnp.int32)
  npad = TOT_E - (ei.shape[1] + N)
  padv = jnp.arange(npad, dtype=jnp.int32) % 16
  src = jnp.concatenate([ei[0], loop, padv]).reshape(NTILES, NBLK, BE)
  dst = jnp.concatenate([ei[1], loop, N + padv]).reshape(NTILES, NBLK, BE)

  w2p = jnp.pad(W2, ((0, 0), (0, 68)))
  b2p = jnp.pad(b2, (0, 68)).reshape(1, 768)
  w3p = jnp.pad(W3, ((0, 68), (0, 56)))
  b3p = jnp.pad(b3, (0, 56)).reshape(1, 256)
  wlp = jnp.pad(Wl, ((0, 56), (0, 0)))
  blp = bl.reshape(1, 10)
  b1p = b1.reshape(1, 1000)
  batchf = jnp.broadcast_to(
      batch.astype(F32)[:, None], (N, NUM_GRAPHS))

  degp = _compute_deg(dst)

  deg_spec = pl.BlockSpec((2, TR, 128), lambda i: (0, i, 0))

  xt = pl.pallas_call(
      _mpre_body,
      grid=(G,),
      in_specs=[pl.BlockSpec((TR, 128), lambda i: (i, 0)), deg_spec],
      out_specs=pl.BlockSpec((TR, 128), lambda i: (i, 0)),
      out_shape=jax.ShapeDtypeStruct((N, 128), F32),
  )(x, degp)

  p0 = _prop(src, dst, [xt])

  t1c = pl.pallas_call(
      _m12_body,
      grid=(G,),
      in_specs=[
          pl.BlockSpec((2, 1, TR, 128), lambda i: (0, 0, i, 0)),
          deg_spec,
          _full((128, 1000)),
          _full((1, 1000)),
          _full((1000, 768)),
      ],
      out_specs=[pl.BlockSpec((TR, 128), lambda i: (i, 0))] * 6,
      out_shape=[jax.ShapeDtypeStruct((N, 128), F32)] * 6,
  )(p0, degp, W1, b1p, w2p)

  p1 = _prop(src, dst, list(t1c))

  t2c = pl.pallas_call(
      _m3_body,
      grid=(G,),
      in_specs=[
          pl.BlockSpec((2, 6, TR, 128), lambda i: (0, 0, i, 0)),
          deg_spec,
          _full((1, 768)),
          _full((768, 256)),
      ],
      out_specs=[pl.BlockSpec((TR, 128), lambda i: (i, 0))] * 2,
      out_shape=[jax.ShapeDtypeStruct((N, 128), F32)] * 2,
  )(p1, degp, b2p, w3p)

  p2 = _prop(src, dst, list(t2c))

  out = pl.pallas_call(
      _m4_body,
      grid=(G,),
      in_specs=[
          pl.BlockSpec((2, 2, TR, 128), lambda i: (0, 0, i, 0)),
          deg_spec,
          _full((1, 256)),
          pl.BlockSpec((TR, NUM_GRAPHS), lambda i: (i, 0)),
          _full((256, 10)),
          _full((1, 10)),
      ],
      out_specs=pl.BlockSpec((NUM_GRAPHS, 10), lambda i: (0, 0)),
      out_shape=jax.ShapeDtypeStruct((NUM_GRAPHS, 10), F32),
      scratch_shapes=[
          pltpu.VMEM((NUM_GRAPHS, 256), F32),
          pltpu.VMEM((NUM_GRAPHS, 8), F32),
      ],
  )(p2, degp, b3p, batchf, wlp, blp)

  return out


# speedup vs baseline: 1.3022x; 1.0054x over previous
import jax
import jax.numpy as jnp
from jax import lax
from jax.experimental import pallas as pl
from jax.experimental.pallas import tpu as pltpu
from jax.experimental.pallas import tpu_sc as plsc

N = 10000
NUM_GRAPHS = 128
NP = 10240
NTILES = 32
NBLK = 42
BE = 128
STRIPE = NP // 16
ZR = 64
TOT_E = NTILES * NBLK * BE
TR = 2000
G = N // TR
F32 = jnp.float32

_MESH = plsc.VectorSubcoreMesh(core_axis_name="c", subcore_axis_name="s",
                               num_cores=2, num_subcores=16)


def _zero_fill(buf, rows, width=128):
  @pl.loop(0, rows)
  def _(r):
    @pl.loop(0, width, step=16)
    def _(f):
      buf[r, pl.ds(f, 16)] = jnp.zeros((16,), F32)


def _deg_body(dst_hbm, out_hbm, dstv, ones_v, zv, acc, sem):
  del sem
  c = lax.axis_index("c")
  s = lax.axis_index("s")
  wid = s * 2 + c
  pltpu.sync_copy(dst_hbm.at[wid], dstv)

  @pl.loop(0, BE)
  def _(r):
    @pl.loop(0, 128, step=16)
    def _(f):
      ones_v[r, pl.ds(f, 16)] = jnp.full((16,), 1.0, F32)

  _zero_fill(zv, ZR)
  base = s * STRIPE
  @pl.loop(0, STRIPE // ZR)
  def _(j):
    pltpu.sync_copy(zv, acc.at[pl.ds(base + j * ZR, ZR)])
  plsc.subcore_barrier()

  @pl.loop(0, NBLK)
  def _(blk):
    pltpu.sync_copy(ones_v, acc.at[dstv.at[blk]], add=True)
  plsc.subcore_barrier()
  pltpu.sync_copy(acc.at[pl.ds(base, STRIPE)], out_hbm.at[c, pl.ds(base, STRIPE)])


def _compute_deg(dst_idx):
  k = pl.kernel(
      _deg_body,
      out_type=jax.ShapeDtypeStruct((2, NP, 128), F32),
      mesh=_MESH,
      scratch_types=[
          pltpu.VMEM((NBLK, BE), jnp.int32),
          pltpu.VMEM((BE, 128), F32),
          pltpu.VMEM((ZR, 128), F32),
          pltpu.VMEM_SHARED((NP, 128), F32),
          pltpu.SemaphoreType.DMA,
      ],
  )
  return k(dst_idx)


def _make_prop_body(nchunks):
  def body(src_hbm, dst_hbm, *rest):
    tabs = rest[:nchunks]
    out_hbm = rest[nchunks]
    srcv, dstv, g0, g1, sem0, sem1, acc = rest[nchunks + 1:]
    c = lax.axis_index("c")
    s = lax.axis_index("s")
    wid = s * 2 + c
    pltpu.sync_copy(src_hbm.at[wid], srcv)
    pltpu.sync_copy(dst_hbm.at[wid], dstv)
    base = s * STRIPE

    for kk in range(nchunks):
      tab = tabs[kk]

      _zero_fill(g0, BE)

      @pl.loop(0, STRIPE // BE)
      def _(j):
        pltpu.sync_copy(g0, acc.at[pl.ds(base + j * BE, BE)])
      plsc.subcore_barrier()

      pltpu.async_copy(tab.at[srcv.at[0]], g0, sem0)
      pltpu.async_copy(tab.at[srcv.at[1]], g1, sem1)

      @pl.loop(0, NBLK, step=2)
      def _(blk):
        pltpu.make_async_copy(tab.at[srcv.at[blk]], g0, sem0).wait()
        pltpu.sync_copy(g0, acc.at[dstv.at[blk]], add=True)

        @pl.when(blk + 2 < NBLK)
        def _():
          pltpu.async_copy(tab.at[srcv.at[blk + 2]], g0, sem0)

        pltpu.make_async_copy(tab.at[srcv.at[blk + 1]], g1, sem1).wait()
        pltpu.sync_copy(g1, acc.at[dstv.at[blk + 1]], add=True)

        @pl.when(blk + 3 < NBLK)
        def _():
          pltpu.async_copy(tab.at[srcv.at[blk + 3]], g1, sem1)

      plsc.subcore_barrier()
      pltpu.sync_copy(acc.at[pl.ds(base, STRIPE)],
                      out_hbm.at[c, kk, pl.ds(base, STRIPE)])
  return body


def _prop(src_idx, dst_idx, tabs):
  nchunks = len(tabs)
  k = pl.kernel(
      _make_prop_body(nchunks),
      out_type=jax.ShapeDtypeStruct((2, nchunks, NP, 128), F32),
      mesh=_MESH,
      scratch_types=[
          pltpu.VMEM((NBLK, BE), jnp.int32),
          pltpu.VMEM((NBLK, BE), jnp.int32),
          pltpu.VMEM((BE, 128), F32),
          pltpu.VMEM((BE, 128), F32),
          pltpu.SemaphoreType.DMA,
          pltpu.SemaphoreType.DMA,
          pltpu.VMEM_SHARED((NP, 128), F32),
      ],
  )
  return k(src_idx, dst_idx, *tabs)


def _dinv_of(deg_blk):
  d = deg_blk[0, :, 0:1] + deg_blk[1, :, 0:1]
  return lax.rsqrt(d)


def _mpre_body(x_ref, deg_ref, o_ref):
  o_ref[...] = x_ref[...] * _dinv_of(deg_ref[...])


def _m12_body(p0_ref, deg_ref, w1_ref, b1_ref, w2_ref, *outs):
  dinv = _dinv_of(deg_ref[...])
  p0 = p0_ref[...]
  p = (p0[0, 0] + p0[1, 0]) * dinv
  h = jnp.maximum(
      jnp.dot(p, w1_ref[...], preferred_element_type=F32) + b1_ref[...], 0.0)
  t = jnp.dot(h, w2_ref[...], preferred_element_type=F32) * dinv
  for kk in range(6):
    outs[kk][...] = t[:, 128 * kk:128 * (kk + 1)]


def _m3_body(p1_ref, deg_ref, b2_ref, w3_ref, oa_ref, ob_ref):
  dinv = _dinv_of(deg_ref[...])
  p1 = p1_ref[...]
  pc = jnp.concatenate([p1[0, kk] + p1[1, kk] for kk in range(6)], axis=1)
  h = jnp.maximum(pc * dinv + b2_ref[...], 0.0)
  t = jnp.dot(h, w3_ref[...], preferred_element_type=F32) * dinv
  oa_ref[...] = t[:, 0:128]
  ob_ref[...] = t[:, 128:256]


def _m4_body(p2_ref, deg_ref, b3_ref, batch_ref, wl_ref, bl_ref, out_ref,
             sums, cnts):
  i = pl.program_id(0)

  @pl.when(i == 0)
  def _():
    sums[...] = jnp.zeros_like(sums)
    cnts[...] = jnp.zeros_like(cnts)

  dinv = _dinv_of(deg_ref[...])
  p2 = p2_ref[...]
  pc = jnp.concatenate([p2[0, 0] + p2[1, 0], p2[0, 1] + p2[1, 1]], axis=1)
  h3 = jnp.maximum(pc * dinv + b3_ref[...], 0.0)
  gf = (batch_ref[...] == lax.broadcasted_iota(
      jnp.int32, (TR, NUM_GRAPHS), 1).astype(F32)).astype(F32)
  dn = (((0,), (0,)), ((), ()))
  sums[...] += lax.dot_general(gf, h3, dn, preferred_element_type=F32)
  cnts[...] += lax.dot_general(gf, jnp.ones((TR, 8), F32), dn,
                               preferred_element_type=F32)

  @pl.when(i == G - 1)
  def _():
    pooled = sums[...] / jnp.maximum(cnts[...][:, 0:1], 1.0)
    out_ref[...] = (jnp.dot(pooled, wl_ref[...], preferred_element_type=F32)
                    + bl_ref[...])


def _full(shape):
  return pl.BlockSpec(shape, lambda i: tuple(0 for _ in shape))


@jax.jit
def kernel(x, edge_index, batch, W1, b1, W2, b2, W3, b3, Wl, bl):
  ei = edge_index.astype(jnp.int32)
  loop = jnp.arange(N, dtype=jnp.int32)
  npad = TOT_E - (ei.shape[1] + N)
  padv = jnp.arange(npad, dtype=jnp.int32) % 16
  src = jnp.concatenate([ei[0], loop, padv]).reshape(NTILES, NBLK, BE)
  dst = jnp.concatenate([ei[1], loop, N + padv]).reshape(NTILES, NBLK, BE)

  w2p = jnp.pad(W2, ((0, 0), (0, 68)))
  b2p = jnp.pad(b2, (0, 68)).reshape(1, 768)
  w3p = jnp.pad(W3, ((0, 68), (0, 56)))
  b3p = jnp.pad(b3, (0, 56)).reshape(1, 256)
  wlp = jnp.pad(Wl, ((0, 56), (0, 0)))
  blp = bl.reshape(1, 10)
  b1p = b1.reshape(1, 1000)
  batchf = jnp.broadcast_to(
      batch.astype(F32)[:, None], (N, NUM_GRAPHS))

  degp = _compute_deg(dst)

  deg_spec = pl.BlockSpec((2, TR, 128), lambda i: (0, i, 0))

  xt = pl.pallas_call(
      _mpre_body,
      grid=(G,),
      in_specs=[pl.BlockSpec((TR, 128), lambda i: (i, 0)), deg_spec],
      out_specs=pl.BlockSpec((TR, 128), lambda i: (i, 0)),
      out_shape=jax.ShapeDtypeStruct((N, 128), F32),
  )(x, degp)

  p0 = _prop(src, dst, [xt])

  t1c = pl.pallas_call(
      _m12_body,
      grid=(G,),
      in_specs=[
          pl.BlockSpec((2, 1, TR, 128), lambda i: (0, 0, i, 0)),
          deg_spec,
          _full((128, 1000)),
          _full((1, 1000)),
          _full((1000, 768)),
      ],
      out_specs=[pl.BlockSpec((TR, 128), lambda i: (i, 0))] * 6,
      out_shape=[jax.ShapeDtypeStruct((N, 128), F32)] * 6,
  )(p0, degp, W1, b1p, w2p)

  p1 = _prop(src, dst, list(t1c))

  t2c = pl.pallas_call(
      _m3_body,
      grid=(G,),
      in_specs=[
          pl.BlockSpec((2, 6, TR, 128), lambda i: (0, 0, i, 0)),
          deg_spec,
          _full((1, 768)),
          _full((768, 256)),
      ],
      out_specs=[pl.BlockSpec((TR, 128), lambda i: (i, 0))] * 2,
      out_shape=[jax.ShapeDtypeStruct((N, 128), F32)] * 2,
  )(p1, degp, b2p, w3p)

  p2 = _prop(src, dst, list(t2c))

  out = pl.pallas_call(
      _m4_body,
      grid=(G,),
      in_specs=[
          pl.BlockSpec((2, 2, TR, 128), lambda i: (0, 0, i, 0)),
          deg_spec,
          _full((1, 256)),
          pl.BlockSpec((TR, NUM_GRAPHS), lambda i: (i, 0)),
          _full((256, 10)),
          _full((1, 10)),
      ],
      out_specs=pl.BlockSpec((NUM_GRAPHS, 10), lambda i: (0, 0)),
      out_shape=jax.ShapeDtypeStruct((NUM_GRAPHS, 10), F32),
      scratch_shapes=[
          pltpu.VMEM((NUM_GRAPHS, 256), F32),
          pltpu.VMEM((NUM_GRAPHS, 8), F32),
      ],
  )(p2, degp, b3p, batchf, wlp, blp)

  return out
